# baseline (speedup 1.0000x reference)
import functools

import jax
import jax.numpy as jnp
from jax import lax
from jax.experimental import pallas as pl
from jax.experimental.pallas import tpu as pltpu
from jax.experimental.pallas import tpu_sc as plsc

WORD_DIM = 128
MAX_NORM = 1.0

NUM_CORES = 2
NUM_SUBCORES = 16
NUM_WORKERS = NUM_CORES * NUM_SUBCORES

CHUNK = 128


def _make_gather(total_rows):
    assert total_rows % (NUM_WORKERS * CHUNK) == 0
    rows_per_w = total_rows // NUM_WORKERS
    nchunk = rows_per_w // CHUNK
    assert nchunk % 2 == 0
    mesh = plsc.VectorSubcoreMesh(core_axis_name="c", subcore_axis_name="s")

    @functools.partial(
        pl.kernel,
        out_type=jax.ShapeDtypeStruct((total_rows, WORD_DIM), jnp.float32),
        mesh=mesh,
        scratch_types=[
            pltpu.VMEM((nchunk, CHUNK), jnp.int32),
            pltpu.VMEM((CHUNK, WORD_DIM), jnp.float32),
            pltpu.VMEM((CHUNK, WORD_DIM), jnp.float32),
            pltpu.SemaphoreType.DMA,
            pltpu.SemaphoreType.DMA,
        ],
    )
    def gather_kernel(idx_hbm, table_hbm, out_hbm, idx_v, rows0, rows1, sem0, sem1):
        wid = lax.axis_index("s") * NUM_CORES + lax.axis_index("c")
        base = wid * rows_per_w
        pltpu.sync_copy(idx_hbm.at[wid], idx_v)

        bufs = (rows0, rows1)
        sems = (sem0, sem1)

        def start(j, b):
            pltpu.async_copy(table_hbm.at[idx_v.at[j]], bufs[b], sems[b])

        def wait(b):
            pltpu.make_async_copy(
                table_hbm.at[idx_v.at[0]], bufs[b], sems[b]
            ).wait()

        def store(j, b):
            pltpu.sync_copy(bufs[b], out_hbm.at[pl.ds(base + j * CHUNK, CHUNK)])

        start(0, 0)

        def body(i, _):
            j0 = 2 * i
            start(j0 + 1, 1)
            wait(0)
            store(j0, 0)

            @pl.when(j0 + 2 < nchunk)
            def _():
                start(j0 + 2, 0)

            wait(1)
            store(j0 + 1, 1)
            return 0

        lax.fori_loop(0, nchunk // 2, body, 0)

    return gather_kernel


def _renorm_body(rows_ref, out_ref):
    x = rows_ref[...]
    norm = jnp.sqrt(jnp.sum(x * x, axis=1, keepdims=True))
    scale = jnp.where(norm > MAX_NORM, MAX_NORM / (norm + 1e-7), 1.0)
    y = x * scale
    out_ref[...] = y.reshape(out_ref.shape)


def _renorm(rows, batch, seq):
    bat_blk = 32
    assert batch % bat_blk == 0
    return pl.pallas_call(
        _renorm_body,
        grid=(batch // bat_blk,),
        in_specs=[pl.BlockSpec((bat_blk * seq, WORD_DIM), lambda i: (i, 0))],
        out_specs=pl.BlockSpec((bat_blk, seq, WORD_DIM), lambda i: (i, 0, 0)),
        out_shape=jax.ShapeDtypeStruct((batch, seq, WORD_DIM), jnp.float32),
    )(rows)


@jax.jit
def kernel(indices, W):
    B, L = indices.shape
    idx = indices.astype(jnp.int32).reshape(NUM_WORKERS, (B * L) // (NUM_WORKERS * CHUNK), CHUNK)
    rows = _make_gather(B * L)(idx, W)
    return _renorm(rows, B, L)


# speedup vs baseline: 1.3715x; 1.3715x over previous
import functools

import jax
import jax.numpy as jnp
from jax import lax
from jax.experimental import pallas as pl
from jax.experimental.pallas import tpu as pltpu
from jax.experimental.pallas import tpu_sc as plsc

WORD_DIM = 128
MAX_NORM = 1.0

NUM_CORES = 2
NUM_SUBCORES = 16
NUM_WORKERS = NUM_CORES * NUM_SUBCORES

BAT_PER_CHUNK = 2


def _prescale_body(w_ref, out_ref):
    x = w_ref[...]
    norm = jnp.sqrt(jnp.sum(x * x, axis=1, keepdims=True))
    scale = jnp.where(norm > MAX_NORM, MAX_NORM / (norm + 1e-7), 1.0)
    out_ref[...] = x * scale


def _prescale(W):
    rows = W.shape[0]
    blk = 2000
    assert rows % blk == 0
    return pl.pallas_call(
        _prescale_body,
        grid=(rows // blk,),
        in_specs=[pl.BlockSpec((blk, WORD_DIM), lambda i: (i, 0))],
        out_specs=pl.BlockSpec((blk, WORD_DIM), lambda i: (i, 0)),
        out_shape=jax.ShapeDtypeStruct((rows, WORD_DIM), jnp.float32),
    )(W)


def _make_gather(batch, seq):
    chunk_rows = BAT_PER_CHUNK * seq
    assert batch % (NUM_WORKERS * BAT_PER_CHUNK) == 0
    bat_per_w = batch // NUM_WORKERS
    nchunk = bat_per_w // BAT_PER_CHUNK
    assert nchunk % 2 == 0
    mesh = plsc.VectorSubcoreMesh(core_axis_name="c", subcore_axis_name="s")

    @functools.partial(
        pl.kernel,
        out_type=jax.ShapeDtypeStruct((batch, seq, WORD_DIM), jnp.float32),
        mesh=mesh,
        scratch_types=[
            pltpu.VMEM((nchunk, chunk_rows), jnp.int32),
            pltpu.VMEM((chunk_rows, WORD_DIM), jnp.float32),
            pltpu.VMEM((chunk_rows, WORD_DIM), jnp.float32),
            pltpu.SemaphoreType.DMA,
            pltpu.SemaphoreType.DMA,
        ],
    )
    def gather_kernel(idx_hbm, table_hbm, out_hbm, idx_v, rows0, rows1, sem0, sem1):
        wid = lax.axis_index("s") * NUM_CORES + lax.axis_index("c")
        base = wid * bat_per_w
        pltpu.sync_copy(idx_hbm.at[wid], idx_v)

        bufs = (rows0, rows1)
        sems = (sem0, sem1)

        def start(j, b):
            pltpu.async_copy(table_hbm.at[idx_v.at[j]], bufs[b], sems[b])

        def wait(b):
            pltpu.make_async_copy(
                table_hbm.at[idx_v.at[0]], bufs[b], sems[b]
            ).wait()

        def store(j, b):
            bat = base + j * BAT_PER_CHUNK
            for k in range(BAT_PER_CHUNK):
                pltpu.sync_copy(bufs[b].at[pl.ds(k * seq, seq)], out_hbm.at[bat + k])

        start(0, 0)

        def body(i, _):
            j0 = 2 * i
            start(j0 + 1, 1)
            wait(0)
            store(j0, 0)

            @pl.when(j0 + 2 < nchunk)
            def _():
                start(j0 + 2, 0)

            wait(1)
            store(j0 + 1, 1)
            return 0

        lax.fori_loop(0, nchunk // 2, body, 0)

    return gather_kernel


@jax.jit
def kernel(indices, W):
    B, L = indices.shape
    scaled = _prescale(W)
    idx = indices.astype(jnp.int32).reshape(
        NUM_WORKERS, (B * L) // (NUM_WORKERS * BAT_PER_CHUNK * L), BAT_PER_CHUNK * L
    )
    return _make_gather(B, L)(idx, scaled)


# speedup vs baseline: 2.0257x; 1.4770x over previous
import functools

import jax
import jax.numpy as jnp
from jax import lax
from jax.experimental import pallas as pl
from jax.experimental.pallas import tpu as pltpu
from jax.experimental.pallas import tpu_sc as plsc

WORD_DIM = 128
MAX_NORM = 1.0

NUM_CORES = 2
NUM_SUBCORES = 16
NUM_WORKERS = NUM_CORES * NUM_SUBCORES

CHUNK = 128


def _prescale_body(w_ref, out_ref):
    x = w_ref[...]
    norm = jnp.sqrt(jnp.sum(x * x, axis=1, keepdims=True))
    scale = jnp.where(norm > MAX_NORM, MAX_NORM / (norm + 1e-7), 1.0)
    out_ref[...] = x * scale


def _prescale(W):
    rows = W.shape[0]
    blk = 2000
    assert rows % blk == 0
    return pl.pallas_call(
        _prescale_body,
        grid=(rows // blk,),
        in_specs=[pl.BlockSpec((blk, WORD_DIM), lambda i: (i, 0))],
        out_specs=pl.BlockSpec((blk, WORD_DIM), lambda i: (i, 0)),
        out_shape=jax.ShapeDtypeStruct((rows, WORD_DIM), jnp.float32),
    )(W)


def _make_gather(batch, seq):
    assert batch % (NUM_WORKERS * CHUNK) == 0 or batch == NUM_WORKERS * CHUNK
    assert batch == NUM_WORKERS * CHUNK
    mesh = plsc.VectorSubcoreMesh(core_axis_name="c", subcore_axis_name="s")

    @functools.partial(
        pl.kernel,
        out_type=jax.ShapeDtypeStruct((seq, batch, WORD_DIM), jnp.float32),
        mesh=mesh,
        scratch_types=[
            pltpu.VMEM((seq, CHUNK), jnp.int32),
            pltpu.VMEM((CHUNK, WORD_DIM), jnp.float32),
            pltpu.VMEM((CHUNK, WORD_DIM), jnp.float32),
            pltpu.SemaphoreType.DMA,
            pltpu.SemaphoreType.DMA,
        ],
    )
    def gather_kernel(idx_hbm, table_hbm, out_hbm, idx_v, rows0, rows1, sem0, sem1):
        wid = lax.axis_index("s") * NUM_CORES + lax.axis_index("c")
        col = wid * CHUNK
        pltpu.sync_copy(idx_hbm.at[wid], idx_v)

        bufs = (rows0, rows1)
        sems = (sem0, sem1)

        def start(t, b):
            pltpu.async_copy(table_hbm.at[idx_v.at[t]], bufs[b], sems[b])

        def wait(b):
            pltpu.make_async_copy(
                table_hbm.at[idx_v.at[0]], bufs[b], sems[b]
            ).wait()

        def store(t, b):
            pltpu.sync_copy(bufs[b], out_hbm.at[t, pl.ds(col, CHUNK)])

        start(0, 0)

        def body(i, _):
            t0 = 2 * i
            start(t0 + 1, 1)
            wait(0)
            store(t0, 0)

            @pl.when(t0 + 2 < seq)
            def _():
                start(t0 + 2, 0)

            wait(1)
            store(t0 + 1, 1)
            return 0

        lax.fori_loop(0, seq // 2, body, 0)

    return gather_kernel


@jax.jit
def kernel(indices, W):
    B, L = indices.shape
    scaled = _prescale(W)
    idx3 = jnp.transpose(
        indices.astype(jnp.int32).reshape(NUM_WORKERS, CHUNK, L), (0, 2, 1)
    )
    out_t = _make_gather(B, L)(idx3, scaled)
    return jnp.transpose(out_t, (1, 0, 2))


# speedup vs baseline: 2.4370x; 1.2030x over previous
import functools

import jax
import jax.numpy as jnp
from jax import lax
from jax.experimental import pallas as pl
from jax.experimental.pallas import tpu as pltpu
from jax.experimental.pallas import tpu_sc as plsc

WORD_DIM = 128
MAX_NORM = 1.0

NUM_CORES = 2
NUM_SUBCORES = 16
NUM_WORKERS = NUM_CORES * NUM_SUBCORES

CHUNK = 128


def _prescale_body(w_ref, out_ref):
    x = w_ref[...]
    ones = jnp.ones((WORD_DIM, WORD_DIM), jnp.float32)
    nsq = jax.lax.dot_general(
        x * x, ones, (((1,), (0,)), ((), ())),
        preferred_element_type=jnp.float32,
    )
    scale = jnp.where(nsq > MAX_NORM * MAX_NORM, jax.lax.rsqrt(nsq), 1.0)
    out_ref[...] = x * scale


def _prescale(W):
    rows = W.shape[0]
    blk = 10000
    assert rows % blk == 0
    return pl.pallas_call(
        _prescale_body,
        grid=(rows // blk,),
        in_specs=[pl.BlockSpec((blk, WORD_DIM), lambda i: (i, 0))],
        out_specs=pl.BlockSpec((blk, WORD_DIM), lambda i: (i, 0)),
        out_shape=jax.ShapeDtypeStruct((rows, WORD_DIM), jnp.float32),
    )(W)


def _make_gather(batch, seq):
    assert batch % (NUM_WORKERS * CHUNK) == 0 or batch == NUM_WORKERS * CHUNK
    assert batch == NUM_WORKERS * CHUNK
    mesh = plsc.VectorSubcoreMesh(core_axis_name="c", subcore_axis_name="s")

    @functools.partial(
        pl.kernel,
        out_type=jax.ShapeDtypeStruct((seq, batch, WORD_DIM), jnp.float32),
        mesh=mesh,
        scratch_types=[
            pltpu.VMEM((seq, CHUNK), jnp.int32),
            pltpu.VMEM((CHUNK, WORD_DIM), jnp.float32),
            pltpu.VMEM((CHUNK, WORD_DIM), jnp.float32),
            pltpu.SemaphoreType.DMA,
            pltpu.SemaphoreType.DMA,
        ],
    )
    def gather_kernel(idx_hbm, table_hbm, out_hbm, idx_v, rows0, rows1, sem0, sem1):
        wid = lax.axis_index("s") * NUM_CORES + lax.axis_index("c")
        col = wid * CHUNK
        pltpu.sync_copy(idx_hbm.at[wid], idx_v)

        bufs = (rows0, rows1)
        sems = (sem0, sem1)

        def start(t, b):
            pltpu.async_copy(table_hbm.at[idx_v.at[t]], bufs[b], sems[b])

        def wait(b):
            pltpu.make_async_copy(
                table_hbm.at[idx_v.at[0]], bufs[b], sems[b]
            ).wait()

        def store(t, b):
            pltpu.sync_copy(bufs[b], out_hbm.at[t, pl.ds(col, CHUNK)])

        start(0, 0)

        def body(i, _):
            t0 = 2 * i
            start(t0 + 1, 1)
            wait(0)
            store(t0, 0)

            @pl.when(t0 + 2 < seq)
            def _():
                start(t0 + 2, 0)

            wait(1)
            store(t0 + 1, 1)
            return 0

        lax.fori_loop(0, seq // 2, body, 0)

    return gather_kernel


@jax.jit
def kernel(indices, W):
    B, L = indices.shape
    scaled = _prescale(W)
    idx3 = jnp.transpose(
        indices.astype(jnp.int32).reshape(NUM_WORKERS, CHUNK, L), (0, 2, 1)
    )
    out_t = _make_gather(B, L)(idx3, scaled)
    return jnp.transpose(out_t, (1, 0, 2))


# speedup vs baseline: 2.4822x; 1.0185x over previous
import functools

import jax
import jax.numpy as jnp
from jax import lax
from jax.experimental import pallas as pl
from jax.experimental.pallas import tpu as pltpu
from jax.experimental.pallas import tpu_sc as plsc

WORD_DIM = 128
MAX_NORM = 1.0

NUM_CORES = 2
NUM_SUBCORES = 16
NUM_WORKERS = NUM_CORES * NUM_SUBCORES

CHUNK = 128


def _prescale_body(w_ref, out_ref):
    x = w_ref[...]
    ones = jnp.ones((WORD_DIM, WORD_DIM), jnp.float32)
    nsq = jax.lax.dot_general(
        x * x, ones, (((1,), (0,)), ((), ())),
        preferred_element_type=jnp.float32,
    )
    scale = jnp.where(nsq > MAX_NORM * MAX_NORM, jax.lax.rsqrt(nsq), 1.0)
    out_ref[...] = x * scale


def _prescale(W):
    rows = W.shape[0]
    blk = 10000
    assert rows % blk == 0
    return pl.pallas_call(
        _prescale_body,
        grid=(rows // blk,),
        in_specs=[pl.BlockSpec((blk, WORD_DIM), lambda i: (i, 0))],
        out_specs=pl.BlockSpec((blk, WORD_DIM), lambda i: (i, 0)),
        out_shape=jax.ShapeDtypeStruct((rows, WORD_DIM), jnp.float32),
    )(W)


def _make_gather(batch, seq):
    assert batch % (NUM_WORKERS * CHUNK) == 0 or batch == NUM_WORKERS * CHUNK
    assert batch == NUM_WORKERS * CHUNK
    mesh = plsc.VectorSubcoreMesh(core_axis_name="c", subcore_axis_name="s")

    NBUF = 5
    assert seq % NBUF == 0

    @functools.partial(
        pl.kernel,
        out_type=jax.ShapeDtypeStruct((seq, batch, WORD_DIM), jnp.float32),
        mesh=mesh,
        scratch_types=[
            pltpu.VMEM((seq, CHUNK), jnp.int32),
            [pltpu.VMEM((CHUNK, WORD_DIM), jnp.float32) for _ in range(NBUF)],
            [pltpu.SemaphoreType.DMA for _ in range(NBUF)],
            [pltpu.SemaphoreType.DMA for _ in range(NBUF)],
        ],
    )
    def gather_kernel(idx_hbm, table_hbm, out_hbm, idx_v, bufs, gsems, ssems):
        wid = lax.axis_index("s") * NUM_CORES + lax.axis_index("c")
        col = wid * CHUNK
        pltpu.sync_copy(idx_hbm.at[wid], idx_v)

        def start_gather(t, b):
            pltpu.async_copy(table_hbm.at[idx_v.at[t]], bufs[b], gsems[b])

        def wait_gather(b):
            pltpu.make_async_copy(
                table_hbm.at[idx_v.at[0]], bufs[b], gsems[b]
            ).wait()

        def start_store(t, b):
            pltpu.async_copy(bufs[b], out_hbm.at[t, pl.ds(col, CHUNK)], ssems[b])

        def wait_store(b):
            pltpu.make_async_copy(
                bufs[b], out_hbm.at[0, pl.ds(col, CHUNK)], ssems[b]
            ).wait()

        for b in range(NBUF - 1):
            start_gather(b, b)

        def body(i, _):
            for k in range(NBUF):
                t = NBUF * i + k
                wait_gather(k)
                start_store(t, k)
                nb = (k + NBUF - 1) % NBUF
                nt = t + NBUF - 1

                @pl.when(nt < seq)
                def _():
                    @pl.when(nt >= NBUF)
                    def _():
                        wait_store(nb)

                    start_gather(nt, nb)

            return 0

        lax.fori_loop(0, seq // NBUF, body, 0)
        for b in range(NBUF):
            wait_store(b)

    return gather_kernel


@jax.jit
def kernel(indices, W):
    B, L = indices.shape
    scaled = _prescale(W)
    idx3 = jnp.transpose(
        indices.astype(jnp.int32).reshape(NUM_WORKERS, CHUNK, L), (0, 2, 1)
    )
    out_t = _make_gather(B, L)(idx3, scaled)
    return jnp.transpose(out_t, (1, 0, 2))
